# baseline (speedup 1.0000x reference)
import functools

import jax
import jax.numpy as jnp
from jax import lax
from jax.experimental import pallas as pl
from jax.experimental.pallas import tpu as pltpu
from jax.experimental.pallas import tpu_sc as plsc

_FIELD_DIM = 100000
_NUM_FIELDS = 26
_TOTAL_ROWS = _FIELD_DIM * _NUM_FIELDS
_BATCH = 16384
_LANES = 16
_NUM_CORES = 2
_NUM_SUBCORES = 16
_NUM_WORKERS = _NUM_CORES * _NUM_SUBCORES
_B_PER_W = _BATCH // _NUM_WORKERS
_VECS = _B_PER_W // _LANES
_GATHER_N = _NUM_FIELDS * _B_PER_W


def _sc_body(xt_hbm, table_hbm, bias_hbm, out_hbm, x_v, idx_v, rows_v,
             out_v, bias_v, sem):
    core = lax.axis_index("c")
    sub = lax.axis_index("s")
    wid = sub * _NUM_CORES + core
    base = wid * _B_PER_W

    pltpu.sync_copy(xt_hbm.at[:, pl.ds(base, _B_PER_W)], x_v)
    pltpu.sync_copy(bias_hbm, bias_v)

    def build(i, _):
        for f in range(_NUM_FIELDS):
            v = x_v[f, pl.ds(i * _LANES, _LANES)]
            idx_v[pl.ds(f * _B_PER_W + i * _LANES, _LANES)] = v + (
                f * _FIELD_DIM)
        return _

    lax.fori_loop(0, _VECS, build, None)

    pltpu.async_copy(table_hbm.at[idx_v], rows_v, sem).wait()

    bias_vec = bias_v[...]

    def reduce(i, _):
        acc = bias_vec
        for f in range(_NUM_FIELDS):
            acc = acc + rows_v[pl.ds(f * _B_PER_W + i * _LANES, _LANES)]
        out_v[pl.ds(i * _LANES, _LANES)] = acc
        return _

    lax.fori_loop(0, _VECS, reduce, None)

    pltpu.sync_copy(out_v, out_hbm.at[pl.ds(base, _B_PER_W)])


@functools.partial(jax.jit, static_argnames=())
def kernel(x, W, bias):
    xt = x.T
    t1 = lax.optimization_barrier(W.reshape(_TOTAL_ROWS // 64, 64))
    table = t1.reshape(_TOTAL_ROWS)
    bias16 = jnp.broadcast_to(bias, (_LANES,))

    mesh = plsc.VectorSubcoreMesh(core_axis_name="c", subcore_axis_name="s")
    run = pl.kernel(
        _sc_body,
        out_type=jax.ShapeDtypeStruct((_BATCH,), jnp.float32),
        mesh=mesh,
        scratch_types=[
            pltpu.VMEM((_NUM_FIELDS, _B_PER_W), jnp.int32),
            pltpu.VMEM((_GATHER_N,), jnp.int32),
            pltpu.VMEM((_GATHER_N,), jnp.float32),
            pltpu.VMEM((_B_PER_W,), jnp.float32),
            pltpu.VMEM((_LANES,), jnp.float32),
            pltpu.SemaphoreType.DMA,
        ],
        compiler_params=pltpu.CompilerParams(use_tc_tiling_on_sc=False),
    )
    return run(xt, table, bias16).reshape(_BATCH, 1)


# speedup vs baseline: 3.3248x; 3.3248x over previous
import functools

import jax
import jax.numpy as jnp
from jax import lax
from jax.experimental import pallas as pl
from jax.experimental.pallas import tpu as pltpu
from jax.experimental.pallas import tpu_sc as plsc

_FIELD_DIM = 100000
_NUM_FIELDS = 26
_TOTAL_ROWS = _FIELD_DIM * _NUM_FIELDS
_PAD_ROWS = 2600960
_BATCH = 16384
_LANES = 16
_NUM_CORES = 2
_NUM_SUBCORES = 16
_NUM_WORKERS = _NUM_CORES * _NUM_SUBCORES
_B_PER_W = _BATCH // _NUM_WORKERS
_VECS = _B_PER_W // _LANES
_GATHER_N = _NUM_FIELDS * _B_PER_W


def _sc_body(xt_hbm, table_hbm, bias_hbm, out_hbm, x_v, idx_v, rows_v,
             out_v, bias_v, sem):
    core = lax.axis_index("c")
    sub = lax.axis_index("s")
    wid = sub * _NUM_CORES + core
    base = wid * _B_PER_W

    pltpu.sync_copy(xt_hbm.at[:, pl.ds(base, _B_PER_W)], x_v)
    pltpu.sync_copy(bias_hbm, bias_v)

    def build(i, _):
        for f in range(_NUM_FIELDS):
            v = x_v[f, pl.ds(i * _LANES, _LANES)]
            idx_v[pl.ds(f * _B_PER_W + i * _LANES, _LANES)] = v + (
                f * _FIELD_DIM)
        return _

    lax.fori_loop(0, _VECS, build, None)

    pltpu.async_copy(table_hbm.at[idx_v], rows_v, sem).wait()

    bias_vec = bias_v[...]

    def reduce(i, _):
        acc = bias_vec
        for f in range(_NUM_FIELDS):
            acc = acc + rows_v[pl.ds(f * _B_PER_W + i * _LANES, _LANES)]
        out_v[pl.ds(i * _LANES, _LANES)] = acc
        return _

    lax.fori_loop(0, _VECS, reduce, None)

    pltpu.sync_copy(out_v, out_hbm.at[pl.ds(base, _B_PER_W)])


@functools.partial(jax.jit, static_argnames=())
def kernel(x, W, bias):
    xt = x.T
    wp = lax.optimization_barrier(jnp.pad(W, ((0, _PAD_ROWS - _TOTAL_ROWS),
                                              (0, 0))))
    table = wp.reshape(_PAD_ROWS)
    bias16 = jnp.broadcast_to(bias, (_LANES,))

    mesh = plsc.VectorSubcoreMesh(core_axis_name="c", subcore_axis_name="s")
    run = pl.kernel(
        _sc_body,
        out_type=jax.ShapeDtypeStruct((_BATCH,), jnp.float32),
        mesh=mesh,
        scratch_types=[
            pltpu.VMEM((_NUM_FIELDS, _B_PER_W), jnp.int32),
            pltpu.VMEM((_GATHER_N,), jnp.int32),
            pltpu.VMEM((_GATHER_N,), jnp.float32),
            pltpu.VMEM((_B_PER_W,), jnp.float32),
            pltpu.VMEM((_LANES,), jnp.float32),
            pltpu.SemaphoreType.DMA,
        ],
        compiler_params=pltpu.CompilerParams(use_tc_tiling_on_sc=False),
    )
    return run(xt, table, bias16).reshape(_BATCH, 1)


# speedup vs baseline: 3.3306x; 1.0017x over previous
import functools

import jax
import jax.numpy as jnp
from jax import lax
from jax.experimental import pallas as pl
from jax.experimental.pallas import tpu as pltpu
from jax.experimental.pallas import tpu_sc as plsc

_FIELD_DIM = 100000
_NUM_FIELDS = 26
_TOTAL_ROWS = _FIELD_DIM * _NUM_FIELDS
_PAD_ROWS = 2600960
_BATCH = 16384
_LANES = 16
_NUM_CORES = 2
_NUM_SUBCORES = 16
_NUM_WORKERS = _NUM_CORES * _NUM_SUBCORES
_B_PER_W = _BATCH // _NUM_WORKERS
_VECS = _B_PER_W // _LANES
_GATHER_N = _NUM_FIELDS * _B_PER_W


_CHUNKS = 4
_ROWS_PER_CHUNK = _B_PER_W // _CHUNKS
_CVECS = _ROWS_PER_CHUNK // _LANES
_CHUNK_N = _NUM_FIELDS * _ROWS_PER_CHUNK


def _sc_body(xt_hbm, table_hbm, bias_hbm, out_hbm, x_v, out_v, bias_v,
             idx_refs, row_refs, sems):
    core = lax.axis_index("c")
    sub = lax.axis_index("s")
    wid = sub * _NUM_CORES + core
    base = wid * _B_PER_W

    pltpu.sync_copy(xt_hbm.at[:, pl.ds(base, _B_PER_W)], x_v)
    pltpu.sync_copy(bias_hbm, bias_v)

    copies = []
    for c in range(_CHUNKS):
        idx_c = idx_refs[c]

        def build(i, _, c=c, idx_c=idx_c):
            row0 = c * _ROWS_PER_CHUNK + i * _LANES
            for f in range(_NUM_FIELDS):
                v = x_v[f, pl.ds(row0, _LANES)]
                idx_c[pl.ds(f * _ROWS_PER_CHUNK + i * _LANES, _LANES)] = (
                    v + f * _FIELD_DIM)
            return _

        lax.fori_loop(0, _CVECS, build, None)
        copies.append(
            pltpu.async_copy(table_hbm.at[idx_c], row_refs[c], sems[c]))

    bias_vec = bias_v[...]
    for c in range(_CHUNKS):
        copies[c].wait()
        rows_c = row_refs[c]

        def reduce(i, _, c=c, rows_c=rows_c):
            acc = bias_vec
            for f in range(_NUM_FIELDS):
                acc = acc + rows_c[
                    pl.ds(f * _ROWS_PER_CHUNK + i * _LANES, _LANES)]
            out_v[pl.ds(c * _ROWS_PER_CHUNK + i * _LANES, _LANES)] = acc
            return _

        lax.fori_loop(0, _CVECS, reduce, None)

    pltpu.sync_copy(out_v, out_hbm.at[pl.ds(base, _B_PER_W)])


@functools.partial(jax.jit, static_argnames=())
def kernel(x, W, bias):
    xt = x.T
    wp = lax.optimization_barrier(jnp.pad(W, ((0, _PAD_ROWS - _TOTAL_ROWS),
                                              (0, 0))))
    table = wp.reshape(_PAD_ROWS)
    bias16 = jnp.broadcast_to(bias, (_LANES,))

    mesh = plsc.VectorSubcoreMesh(core_axis_name="c", subcore_axis_name="s")
    run = pl.kernel(
        _sc_body,
        out_type=jax.ShapeDtypeStruct((_BATCH,), jnp.float32),
        mesh=mesh,
        scratch_types=[
            pltpu.VMEM((_NUM_FIELDS, _B_PER_W), jnp.int32),
            pltpu.VMEM((_B_PER_W,), jnp.float32),
            pltpu.VMEM((_LANES,), jnp.float32),
            [pltpu.VMEM((_CHUNK_N,), jnp.int32) for _ in range(_CHUNKS)],
            [pltpu.VMEM((_CHUNK_N,), jnp.float32) for _ in range(_CHUNKS)],
            [pltpu.SemaphoreType.DMA for _ in range(_CHUNKS)],
        ],
        compiler_params=pltpu.CompilerParams(use_tc_tiling_on_sc=False),
    )
    return run(xt, table, bias16).reshape(_BATCH, 1)


# speedup vs baseline: 3.5180x; 1.0563x over previous
import functools

import jax
import jax.numpy as jnp
from jax import lax
from jax.experimental import pallas as pl
from jax.experimental.pallas import tpu as pltpu
from jax.experimental.pallas import tpu_sc as plsc

_FIELD_DIM = 100000
_NUM_FIELDS = 26
_TOTAL_ROWS = _FIELD_DIM * _NUM_FIELDS
_PAD_ROWS = 2600960
_BATCH = 16384
_LANES = 16
_NUM_CORES = 2
_NUM_SUBCORES = 16
_NUM_WORKERS = _NUM_CORES * _NUM_SUBCORES
_B_PER_W = _BATCH // _NUM_WORKERS
_VECS = _B_PER_W // _LANES
_GATHER_N = _NUM_FIELDS * _B_PER_W


_CHUNKS = 4
_ROWS_PER_CHUNK = _B_PER_W // _CHUNKS
_CVECS = _ROWS_PER_CHUNK // _LANES
_CHUNK_N = _NUM_FIELDS * _ROWS_PER_CHUNK


def _sc_body(xt_hbm, table_hbm, bias_hbm, out_hbm, x_v, out_v, bias_v,
             idx_refs, row_refs, sems):
    core = lax.axis_index("c")
    sub = lax.axis_index("s")
    wid = sub * _NUM_CORES + core
    base = wid * _B_PER_W

    pltpu.sync_copy(xt_hbm.at[:, pl.ds(base, _B_PER_W)], x_v)
    pltpu.sync_copy(bias_hbm, bias_v)

    copies = []
    for c in range(_CHUNKS):
        idx_c = idx_refs[c]

        def build(i, _, c=c, idx_c=idx_c):
            row0 = c * _ROWS_PER_CHUNK + i * _LANES
            for f in range(_NUM_FIELDS):
                v = x_v[f, pl.ds(row0, _LANES)]
                idx_c[pl.ds(f * _ROWS_PER_CHUNK + i * _LANES, _LANES)] = (
                    v + f * _FIELD_DIM)
            return _

        lax.fori_loop(0, _CVECS, build, None)
        copies.append(
            pltpu.async_copy(table_hbm.at[idx_c], row_refs[c], sems[c]))

    bias_vec = bias_v[...]
    for c in range(_CHUNKS):
        copies[c].wait()
        rows_c = row_refs[c]

        def reduce(i, _, c=c, rows_c=rows_c):
            acc = bias_vec
            for f in range(_NUM_FIELDS):
                acc = acc + rows_c[
                    pl.ds(f * _ROWS_PER_CHUNK + i * _LANES, _LANES)]
            out_v[pl.ds(c * _ROWS_PER_CHUNK + i * _LANES, _LANES)] = acc
            return _

        lax.fori_loop(0, _CVECS, reduce, None)

    pltpu.sync_copy(out_v, out_hbm.at[pl.ds(base, _B_PER_W)])


@functools.partial(jax.jit, static_argnames=())
def kernel(x, W, bias):
    xt = x.T
    wp = lax.optimization_barrier(jnp.pad(W, ((0, _PAD_ROWS - _TOTAL_ROWS),
                                              (0, 0))))
    table = wp.reshape(_PAD_ROWS)
    bias16 = jnp.broadcast_to(bias, (_LANES,))

    mesh = plsc.VectorSubcoreMesh(core_axis_name="c", subcore_axis_name="s")
    run = pl.kernel(
        _sc_body,
        out_type=jax.ShapeDtypeStruct((_BATCH,), jnp.float32),
        mesh=mesh,
        scratch_types=[
            pltpu.VMEM((_NUM_FIELDS, _B_PER_W), jnp.int32),
            pltpu.VMEM((_B_PER_W,), jnp.float32),
            pltpu.VMEM((_LANES,), jnp.float32),
            [pltpu.VMEM((_CHUNK_N,), jnp.int32) for _ in range(_CHUNKS)],
            [pltpu.VMEM((_CHUNK_N,), jnp.float32) for _ in range(_CHUNKS)],
            [pltpu.SemaphoreType.DMA for _ in range(_CHUNKS)],
        ],
        compiler_params=pltpu.CompilerParams(),
    )
    return run(xt, table, bias16).reshape(_BATCH, 1)
